# baseline (speedup 1.0000x reference)
import jax
import jax.numpy as jnp
from jax.experimental import pallas as pl
from jax.experimental.pallas import tpu as pltpu

N, H, E, F = 512, 1024, 8, 256


def _moe_body(x_ref, wr_ref, wg_ref, up_hbm, down_hbm, out_ref, eid_ref,
              ubuf, dbuf, act_ref, usem, dsem):
    for e in range(E):
        pltpu.make_async_copy(up_hbm.at[e], ubuf.at[e], usem.at[e]).start()
        pltpu.make_async_copy(down_hbm.at[e], dbuf.at[pl.ds(e * F, F)],
                              dsem.at[e]).start()

    x = x_ref[...]
    logits = jax.lax.dot_general(
        x, wr_ref[...], (((1,), (1,)), ((), ())),
        preferred_element_type=jnp.float32)
    eid = jnp.argmax(logits, axis=1, keepdims=True).astype(jnp.int32)
    eid_ref[...] = eid
    g = jax.lax.dot_general(
        x, wg_ref[...], (((1,), (1,)), ((), ())),
        precision=jax.lax.Precision.HIGHEST,
        preferred_element_type=jnp.float32)
    gate = jax.nn.sigmoid(g)

    for e in range(E):
        pltpu.make_async_copy(up_hbm.at[e], ubuf.at[e], usem.at[e]).wait()
        h = jax.lax.dot_general(
            x, ubuf[e], (((1,), (0,)), ((), ())),
            preferred_element_type=jnp.float32)
        act = h * jax.nn.sigmoid(h)
        act_ref[:, e * F:(e + 1) * F] = jnp.where(eid == e, act, 0.0)

    for e in range(E):
        pltpu.make_async_copy(down_hbm.at[e], dbuf.at[pl.ds(e * F, F)],
                              dsem.at[e]).wait()
    expert_out = jax.lax.dot_general(
        act_ref[...], dbuf[...], (((1,), (0,)), ((), ())),
        preferred_element_type=jnp.float32)
    out_ref[...] = x + gate * expert_out


def kernel(x, W_router, W_gate, up, down):
    out, eid = pl.pallas_call(
        _moe_body,
        in_specs=[
            pl.BlockSpec(memory_space=pltpu.VMEM),
            pl.BlockSpec(memory_space=pltpu.VMEM),
            pl.BlockSpec(memory_space=pltpu.VMEM),
            pl.BlockSpec(memory_space=pl.ANY),
            pl.BlockSpec(memory_space=pl.ANY),
        ],
        out_specs=[
            pl.BlockSpec(memory_space=pltpu.VMEM),
            pl.BlockSpec(memory_space=pltpu.VMEM),
        ],
        out_shape=[
            jax.ShapeDtypeStruct((N, H), jnp.float32),
            jax.ShapeDtypeStruct((N, 1), jnp.int32),
        ],
        scratch_shapes=[
            pltpu.VMEM((E, H, F), jnp.float32),
            pltpu.VMEM((E * F, H), jnp.float32),
            pltpu.VMEM((N, E * F), jnp.float32),
            pltpu.SemaphoreType.DMA((E,)),
            pltpu.SemaphoreType.DMA((E,)),
        ],
    )(x, W_router, W_gate, up, down)
    return (out, eid.reshape(N))
